# initial kernel scaffold (unmeasured)
import jax
import jax.numpy as jnp
from jax import lax
from jax.experimental import pallas as pl
from jax.experimental.pallas import tpu as pltpu

N_DEV = 32


def kernel(x, Win0, Wout0, Win1, Wout1, Win2, Wout2):
    m, d = x.shape

    def body(x_ref, win0_ref, wout0_ref, win1_ref, wout1_ref, win2_ref,
             wout2_ref, out_ref, buf, send_sems, recv_sems, credit_sem):
        me = lax.axis_index("i")
        left = lax.rem(me - 1 + N_DEV, N_DEV)
        right = lax.rem(me + 1, N_DEV)

        barrier = pltpu.get_barrier_semaphore()
        for nbr in (left, right):
            pl.semaphore_signal(barrier, inc=1, device_id=(nbr,),
                                device_id_type=pl.DeviceIdType.MESH)
        pl.semaphore_wait(barrier, 2)

        buf[0, 0] = x_ref[...]

        weights = ((win0_ref, wout0_ref), (win1_ref, wout1_ref),
                   (win2_ref, wout2_ref))

        for l in range(3):
            win_ref, wout_ref = weights[l]

            def step(s, carry, l=l, win_ref=win_ref, wout_ref=wout_ref):
                p = lax.rem(s, 2)
                q = lax.rem(s + 1, 2)

                recv = pltpu.make_async_remote_copy(
                    src_ref=buf.at[q],
                    dst_ref=buf.at[p],
                    send_sem=send_sems.at[q],
                    recv_sem=recv_sems.at[p],
                    device_id=(left,),
                    device_id_type=pl.DeviceIdType.MESH,
                )
                if l == 0:
                    @pl.when(s >= 1)
                    def _():
                        recv.wait_recv()
                else:
                    recv.wait_recv()

                if l > 0:
                    @pl.when(s == 0)
                    def _():
                        buf[0, 0] = buf[0, 1]

                xj = buf[p, 0]
                h = jnp.maximum(
                    jnp.dot(xj, win_ref[...],
                            preferred_element_type=jnp.float32), 0.0)
                c = jnp.dot(h, wout_ref[...],
                            preferred_element_type=jnp.float32)
                prev = buf[p, 1]
                buf[p, 1] = jnp.where(s == 0, c, prev + c)

                if l == 0:
                    @pl.when(s >= 2)
                    def _():
                        pl.semaphore_wait(credit_sem, 1)
                else:
                    pl.semaphore_wait(credit_sem, 1)

                send = pltpu.make_async_remote_copy(
                    src_ref=buf.at[p],
                    dst_ref=buf.at[q],
                    send_sem=send_sems.at[p],
                    recv_sem=recv_sems.at[q],
                    device_id=(right,),
                    device_id_type=pl.DeviceIdType.MESH,
                )
                send.start()
                send.wait_send()

                def grant():
                    pl.semaphore_signal(credit_sem, inc=1, device_id=(left,),
                                        device_id_type=pl.DeviceIdType.MESH)
                if l == 0:
                    pl.when(s >= 1)(grant)
                elif l == 2:
                    pl.when(s <= N_DEV - 2)(grant)
                else:
                    grant()

                return carry

            lax.fori_loop(0, N_DEV, step, 0)

        final_recv = pltpu.make_async_remote_copy(
            src_ref=buf.at[1],
            dst_ref=buf.at[0],
            send_sem=send_sems.at[1],
            recv_sem=recv_sems.at[0],
            device_id=(left,),
            device_id_type=pl.DeviceIdType.MESH,
        )
        final_recv.wait_recv()
        out_ref[...] = buf[0, 1]

    return pl.pallas_call(
        body,
        out_shape=jax.ShapeDtypeStruct((m, d), jnp.float32),
        in_specs=[pl.BlockSpec(memory_space=pltpu.VMEM)] * 7,
        out_specs=pl.BlockSpec(memory_space=pltpu.VMEM),
        scratch_shapes=[
            pltpu.VMEM((2, 2, m, d), jnp.float32),
            pltpu.SemaphoreType.DMA((2,)),
            pltpu.SemaphoreType.DMA((2,)),
            pltpu.SemaphoreType.REGULAR,
        ],
        compiler_params=pltpu.CompilerParams(collective_id=0),
    )(x, Win0, Wout0, Win1, Wout1, Win2, Wout2)


# baseline (device time: 1004104 ns/iter reference)
import jax
import jax.numpy as jnp
from jax import lax
from jax.experimental import pallas as pl
from jax.experimental.pallas import tpu as pltpu

N_DEV = 32


def kernel(x, Win0, Wout0, Win1, Wout1, Win2, Wout2):
    m, d = x.shape

    def body(x_ref, win0_ref, wout0_ref, win1_ref, wout1_ref, win2_ref,
             wout2_ref, out_ref, buf, send_sems, recv_sems, credit_sem):
        me = lax.axis_index("i")
        left = lax.rem(me - 1 + N_DEV, N_DEV)
        right = lax.rem(me + 1, N_DEV)

        barrier = pltpu.get_barrier_semaphore()
        for nbr in (left, right):
            pl.semaphore_signal(barrier, inc=1, device_id=(nbr,),
                                device_id_type=pl.DeviceIdType.MESH)
        pl.semaphore_wait(barrier, 2)

        buf[0, 0] = x_ref[...]

        weights = ((win0_ref, wout0_ref), (win1_ref, wout1_ref),
                   (win2_ref, wout2_ref))

        for l in range(3):
            win_ref, wout_ref = weights[l]

            def step(s, carry, l=l, win_ref=win_ref, wout_ref=wout_ref):
                p = lax.rem(s, 2)
                q = lax.rem(s + 1, 2)

                recv = pltpu.make_async_remote_copy(
                    src_ref=buf.at[q],
                    dst_ref=buf.at[p],
                    send_sem=send_sems.at[q],
                    recv_sem=recv_sems.at[p],
                    device_id=(left,),
                    device_id_type=pl.DeviceIdType.MESH,
                )
                if l == 0:
                    @pl.when(s >= 1)
                    def _():
                        recv.wait_recv()
                else:
                    recv.wait_recv()

                if l > 0:
                    @pl.when(s == 0)
                    def _():
                        buf[0, 0] = buf[0, 1]

                xj = buf[p, 0]
                h = jnp.maximum(
                    jnp.dot(xj, win_ref[...],
                            preferred_element_type=jnp.float32), 0.0)
                c = jnp.dot(h, wout_ref[...],
                            preferred_element_type=jnp.float32)
                prev = buf[p, 1]
                buf[p, 1] = jnp.where(s == 0, c, prev + c)

                if l == 0:
                    @pl.when(s >= 2)
                    def _():
                        pl.semaphore_wait(credit_sem, 1)
                else:
                    pl.semaphore_wait(credit_sem, 1)

                send = pltpu.make_async_remote_copy(
                    src_ref=buf.at[p],
                    dst_ref=buf.at[q],
                    send_sem=send_sems.at[p],
                    recv_sem=recv_sems.at[q],
                    device_id=(right,),
                    device_id_type=pl.DeviceIdType.MESH,
                )
                send.start()
                send.wait_send()

                def grant():
                    pl.semaphore_signal(credit_sem, inc=1, device_id=(left,),
                                        device_id_type=pl.DeviceIdType.MESH)
                if l == 0:
                    pl.when(s >= 1)(grant)
                elif l == 2:
                    pl.when(s <= N_DEV - 2)(grant)
                else:
                    grant()

                return carry

            lax.fori_loop(0, N_DEV, step, 0)

        final_recv = pltpu.make_async_remote_copy(
            src_ref=buf.at[1],
            dst_ref=buf.at[0],
            send_sem=send_sems.at[1],
            recv_sem=recv_sems.at[0],
            device_id=(left,),
            device_id_type=pl.DeviceIdType.MESH,
        )
        final_recv.wait_recv()
        out_ref[...] = buf[0, 1]

    return pl.pallas_call(
        body,
        out_shape=jax.ShapeDtypeStruct((m, d), jnp.float32),
        in_specs=[pl.BlockSpec(memory_space=pltpu.VMEM)] * 7,
        out_specs=pl.BlockSpec(memory_space=pltpu.VMEM),
        scratch_shapes=[
            pltpu.VMEM((2, 2, m, d), jnp.float32),
            pltpu.SemaphoreType.DMA((2,)),
            pltpu.SemaphoreType.DMA((2,)),
            pltpu.SemaphoreType.REGULAR,
        ],
        compiler_params=pltpu.CompilerParams(
            collective_id=0,
            vmem_limit_bytes=100 * 1024 * 1024,
        ),
    )(x, Win0, Wout0, Win1, Wout1, Win2, Wout2)


# device time: 797264 ns/iter; 1.2594x vs baseline; 1.2594x over previous
import jax
import jax.numpy as jnp
from jax import lax
from jax.experimental import pallas as pl
from jax.experimental.pallas import tpu as pltpu

N_DEV = 32


def kernel(x, Win0, Wout0, Win1, Wout1, Win2, Wout2):
    m, d = x.shape

    def body(x_ref, win0_ref, wout0_ref, win1_ref, wout1_ref, win2_ref,
             wout2_ref, out_ref, xbuf, accbuf, x_send_sems, x_recv_sems,
             acc_send_sems, acc_recv_sems, x_credit, acc_credit):
        me = lax.axis_index("i")
        left = lax.rem(me - 1 + N_DEV, N_DEV)
        right = lax.rem(me + 1, N_DEV)

        barrier = pltpu.get_barrier_semaphore()
        for nbr in (left, right):
            pl.semaphore_signal(barrier, inc=1, device_id=(nbr,),
                                device_id_type=pl.DeviceIdType.MESH)
        pl.semaphore_wait(barrier, 2)

        xbuf[0] = x_ref[...]

        weights = ((win0_ref, wout0_ref), (win1_ref, wout1_ref),
                   (win2_ref, wout2_ref))

        for l in range(3):
            win_ref, wout_ref = weights[l]

            def step(s, carry, l=l, win_ref=win_ref, wout_ref=wout_ref):
                px = lax.rem(s, 4)
                pxn = lax.rem(s + 1, 4)
                pa = lax.rem(s, 2)
                pan = lax.rem(s + 1, 2)

                x_recv = pltpu.make_async_remote_copy(
                    src_ref=xbuf.at[pxn],
                    dst_ref=xbuf.at[px],
                    send_sem=x_send_sems.at[pxn],
                    recv_sem=x_recv_sems.at[px],
                    device_id=(left,),
                    device_id_type=pl.DeviceIdType.MESH,
                )
                if l == 0:
                    @pl.when(s >= 1)
                    def _():
                        x_recv.wait_recv()
                else:
                    x_recv.wait_recv()

                x_send = pltpu.make_async_remote_copy(
                    src_ref=xbuf.at[px],
                    dst_ref=xbuf.at[pxn],
                    send_sem=x_send_sems.at[px],
                    recv_sem=x_recv_sems.at[pxn],
                    device_id=(right,),
                    device_id_type=pl.DeviceIdType.MESH,
                )

                @pl.when(s <= N_DEV - 2)
                def _():
                    if l == 0:
                        @pl.when(s >= 3)
                        def _():
                            pl.semaphore_wait(x_credit, 1)
                    else:
                        pl.semaphore_wait(x_credit, 1)
                    x_send.start()

                xj = xbuf[px]
                h = jnp.maximum(
                    jnp.dot(xj, win_ref[...],
                            preferred_element_type=jnp.float32), 0.0)
                c = jnp.dot(h, wout_ref[...],
                            preferred_element_type=jnp.float32)

                @pl.when(s <= N_DEV - 2)
                def _():
                    x_send.wait_send()

                def grant_x():
                    pl.semaphore_signal(x_credit, inc=1, device_id=(left,),
                                        device_id_type=pl.DeviceIdType.MESH)
                if l < 2:
                    grant_x()
                else:
                    pl.when(s <= N_DEV - 4)(grant_x)

                prev_acc = pltpu.make_async_remote_copy(
                    src_ref=accbuf.at[pan],
                    dst_ref=accbuf.at[pa],
                    send_sem=acc_send_sems.at[pan],
                    recv_sem=acc_recv_sems.at[pa],
                    device_id=(right,),
                    device_id_type=pl.DeviceIdType.MESH,
                )
                if l == 0:
                    @pl.when(s >= 1)
                    def _():
                        prev_acc.wait_send()
                else:
                    prev_acc.wait_send()

                def grant_acc():
                    pl.semaphore_signal(acc_credit, inc=1, device_id=(left,),
                                        device_id_type=pl.DeviceIdType.MESH)
                if l == 0:
                    pl.when((s >= 1) & (s <= N_DEV - 2))(grant_acc)
                else:
                    pl.when(s <= N_DEV - 2)(grant_acc)

                acc_recv = pltpu.make_async_remote_copy(
                    src_ref=accbuf.at[pan],
                    dst_ref=accbuf.at[pa],
                    send_sem=acc_send_sems.at[pan],
                    recv_sem=acc_recv_sems.at[pa],
                    device_id=(left,),
                    device_id_type=pl.DeviceIdType.MESH,
                )

                @pl.when(s >= 1)
                def _():
                    acc_recv.wait_recv()

                prev = accbuf[pa]
                accbuf[pa] = jnp.where(s == 0, c, prev + c)

                @pl.when(s <= N_DEV - 2)
                def _():
                    if l == 0:
                        @pl.when(s >= 1)
                        def _():
                            pl.semaphore_wait(acc_credit, 1)
                    else:
                        pl.semaphore_wait(acc_credit, 1)
                    acc_send = pltpu.make_async_remote_copy(
                        src_ref=accbuf.at[pa],
                        dst_ref=accbuf.at[pan],
                        send_sem=acc_send_sems.at[pa],
                        recv_sem=acc_recv_sems.at[pan],
                        device_id=(right,),
                        device_id_type=pl.DeviceIdType.MESH,
                    )
                    acc_send.start()

                @pl.when(s == N_DEV - 1)
                def _():
                    pl.semaphore_wait(x_credit, 1)
                    final_send = pltpu.make_async_remote_copy(
                        src_ref=accbuf.at[pa],
                        dst_ref=xbuf.at[pxn],
                        send_sem=acc_send_sems.at[pa],
                        recv_sem=x_recv_sems.at[pxn],
                        device_id=(right,),
                        device_id_type=pl.DeviceIdType.MESH,
                    )
                    final_send.start()

                return carry

            lax.fori_loop(0, N_DEV, step, 0)

        final_recv = pltpu.make_async_remote_copy(
            src_ref=xbuf.at[1],
            dst_ref=xbuf.at[0],
            send_sem=x_send_sems.at[1],
            recv_sem=x_recv_sems.at[0],
            device_id=(left,),
            device_id_type=pl.DeviceIdType.MESH,
        )
        final_recv.wait_recv()
        out_ref[...] = xbuf[0]

        last_acc = pltpu.make_async_remote_copy(
            src_ref=accbuf.at[1],
            dst_ref=xbuf.at[0],
            send_sem=acc_send_sems.at[1],
            recv_sem=x_recv_sems.at[0],
            device_id=(right,),
            device_id_type=pl.DeviceIdType.MESH,
        )
        last_acc.wait_send()

    return pl.pallas_call(
        body,
        out_shape=jax.ShapeDtypeStruct((m, d), jnp.float32),
        in_specs=[pl.BlockSpec(memory_space=pltpu.VMEM)] * 7,
        out_specs=pl.BlockSpec(memory_space=pltpu.VMEM),
        scratch_shapes=[
            pltpu.VMEM((4, m, d), jnp.float32),
            pltpu.VMEM((2, m, d), jnp.float32),
            pltpu.SemaphoreType.DMA((4,)),
            pltpu.SemaphoreType.DMA((4,)),
            pltpu.SemaphoreType.DMA((2,)),
            pltpu.SemaphoreType.DMA((2,)),
            pltpu.SemaphoreType.REGULAR,
            pltpu.SemaphoreType.REGULAR,
        ],
        compiler_params=pltpu.CompilerParams(
            collective_id=0,
            vmem_limit_bytes=100 * 1024 * 1024,
        ),
    )(x, Win0, Wout0, Win1, Wout1, Win2, Wout2)


# device time: 673029 ns/iter; 1.4919x vs baseline; 1.1846x over previous
import jax
import jax.numpy as jnp
from jax import lax
from jax.experimental import pallas as pl
from jax.experimental.pallas import tpu as pltpu

N_DEV = 32
MESH = pl.DeviceIdType.MESH


def kernel(x, Win0, Wout0, Win1, Wout1, Win2, Wout2):
    m, d = x.shape

    def body(x_ref, win0_ref, wout0_ref, win1_ref, wout1_ref, win2_ref,
             wout2_ref, out_ref, xbuf, accbuf, cbuf, x_send_sems,
             x_recv_sems, acc_send_sems, acc_recv_sems, x_credit,
             acc_credit):
        me = lax.axis_index("i")
        left = lax.rem(me - 1 + N_DEV, N_DEV)
        right = lax.rem(me + 1, N_DEV)

        weights = ((win0_ref, wout0_ref), (win1_ref, wout1_ref),
                   (win2_ref, wout2_ref))

        def mlp(l, xval):
            win_ref, wout_ref = weights[l]
            h = jnp.maximum(
                jnp.dot(xval, win_ref[...],
                        preferred_element_type=jnp.float32), 0.0)
            return jnp.dot(h, wout_ref[...],
                           preferred_element_type=jnp.float32)

        def x_fwd(src_slot, dst_slot):
            return pltpu.make_async_remote_copy(
                src_ref=xbuf.at[src_slot], dst_ref=xbuf.at[dst_slot],
                send_sem=x_send_sems.at[src_slot],
                recv_sem=x_recv_sems.at[dst_slot],
                device_id=(right,), device_id_type=MESH)

        def x_rcv(slot):
            return pltpu.make_async_remote_copy(
                src_ref=xbuf.at[slot], dst_ref=xbuf.at[slot],
                send_sem=x_send_sems.at[slot],
                recv_sem=x_recv_sems.at[slot],
                device_id=(left,), device_id_type=MESH)

        def acc_snd(slot, nslot):
            return pltpu.make_async_remote_copy(
                src_ref=accbuf.at[slot], dst_ref=accbuf.at[nslot],
                send_sem=acc_send_sems.at[slot],
                recv_sem=acc_recv_sems.at[nslot],
                device_id=(right,), device_id_type=MESH)

        def acc_rcv(slot):
            return pltpu.make_async_remote_copy(
                src_ref=accbuf.at[slot], dst_ref=accbuf.at[slot],
                send_sem=acc_send_sems.at[slot],
                recv_sem=acc_recv_sems.at[slot],
                device_id=(left,), device_id_type=MESH)

        def grant(sem):
            pl.semaphore_signal(sem, inc=1, device_id=(left,),
                                device_id_type=MESH)

        barrier = pltpu.get_barrier_semaphore()
        for nbr in (left, right):
            pl.semaphore_signal(barrier, inc=1, device_id=(nbr,),
                                device_id_type=MESH)
        pl.semaphore_wait(barrier, 2)

        xbuf[0] = x_ref[...]
        f0 = x_fwd(0, 1)
        f0.start()
        cbuf[0] = mlp(0, xbuf[0])
        f0.wait_send()
        grant(x_credit)

        for l in range(3):

            def step(s, carry, l=l):
                p4 = lax.rem(s, 4)
                n4 = lax.rem(s + 1, 4)
                nn4 = lax.rem(s + 2, 4)
                p2 = lax.rem(s, 2)
                n2 = lax.rem(s + 1, 2)

                x_rcv(n4).wait_recv()

                fwd = x_fwd(n4, nn4)

                @pl.when(s <= N_DEV - 3)
                def _():
                    if l == 0:
                        @pl.when(s >= 2)
                        def _():
                            pl.semaphore_wait(x_credit, 1)
                    else:
                        pl.semaphore_wait(x_credit, 1)
                    fwd.start()

                def acc_house():
                    acc_snd(n2, p2).wait_send()
                    grant(acc_credit)
                    pl.semaphore_wait(acc_credit, 1)
                if l == 0:
                    pl.when(s >= 1)(acc_house)
                else:
                    acc_house()

                @pl.when(s >= 1)
                def _():
                    acc_rcv(p2).wait_recv()
                prev = accbuf[p2]
                accbuf[p2] = jnp.where(s == 0, cbuf[p2], prev + cbuf[p2])

                acc_snd(p2, n2).start()

                cbuf[n2] = mlp(l, xbuf[n4])

                @pl.when(s <= N_DEV - 3)
                def _():
                    fwd.wait_send()

                def grant_x():
                    grant(x_credit)
                if l < 2:
                    grant_x()
                else:
                    pl.when(s <= N_DEV - 5)(grant_x)

                return carry

            lax.fori_loop(0, N_DEV - 1, step, 0)

            acc_snd(0, 1).wait_send()
            acc_rcv(1).wait_recv()
            accbuf[1] = accbuf[1] + cbuf[1]
            pl.semaphore_wait(x_credit, 1)
            home = pltpu.make_async_remote_copy(
                src_ref=accbuf.at[1], dst_ref=xbuf.at[0],
                send_sem=acc_send_sems.at[1], recv_sem=x_recv_sems.at[0],
                device_id=(right,), device_id_type=MESH)
            home.start()

            if l < 2:
                x_rcv(0).wait_recv()
                pl.semaphore_wait(x_credit, 1)
                fb = x_fwd(0, 1)
                fb.start()
                cbuf[0] = mlp(l + 1, xbuf[0])
                fb.wait_send()
                grant(x_credit)

        x_rcv(0).wait_recv()
        out_ref[...] = xbuf[0]
        last = pltpu.make_async_remote_copy(
            src_ref=accbuf.at[1], dst_ref=xbuf.at[0],
            send_sem=acc_send_sems.at[1], recv_sem=x_recv_sems.at[0],
            device_id=(right,), device_id_type=MESH)
        last.wait_send()

    return pl.pallas_call(
        body,
        out_shape=jax.ShapeDtypeStruct((m, d), jnp.float32),
        in_specs=[pl.BlockSpec(memory_space=pltpu.VMEM)] * 7,
        out_specs=pl.BlockSpec(memory_space=pltpu.VMEM),
        scratch_shapes=[
            pltpu.VMEM((4, m, d), jnp.float32),
            pltpu.VMEM((2, m, d), jnp.float32),
            pltpu.VMEM((2, m, d), jnp.float32),
            pltpu.SemaphoreType.DMA((4,)),
            pltpu.SemaphoreType.DMA((4,)),
            pltpu.SemaphoreType.DMA((2,)),
            pltpu.SemaphoreType.DMA((2,)),
            pltpu.SemaphoreType.REGULAR,
            pltpu.SemaphoreType.REGULAR,
        ],
        compiler_params=pltpu.CompilerParams(
            collective_id=0,
            vmem_limit_bytes=100 * 1024 * 1024,
        ),
    )(x, Win0, Wout0, Win1, Wout1, Win2, Wout2)


# device time: 668926 ns/iter; 1.5011x vs baseline; 1.0061x over previous
import jax
import jax.numpy as jnp
from jax import lax
from jax.experimental import pallas as pl
from jax.experimental.pallas import tpu as pltpu

N_DEV = 32
MESH = pl.DeviceIdType.MESH


def kernel(x, Win0, Wout0, Win1, Wout1, Win2, Wout2):
    m, d = x.shape

    def body(x_ref, win0_ref, wout0_ref, win1_ref, wout1_ref, win2_ref,
             wout2_ref, out_ref, xbuf, accbuf, cbuf, homebuf, x_send_sems,
             x_recv_sems, acc_send_sems, acc_recv_sems, home_sem, x_credit,
             acc_credit):
        me = lax.axis_index("i")
        left = lax.rem(me - 1 + N_DEV, N_DEV)
        right = lax.rem(me + 1, N_DEV)

        weights = ((win0_ref, wout0_ref), (win1_ref, wout1_ref),
                   (win2_ref, wout2_ref))

        def mlp(l, xval):
            win_ref, wout_ref = weights[l]
            xval = xval.astype(jnp.float32)
            h = jnp.maximum(
                jnp.dot(xval, win_ref[...],
                        preferred_element_type=jnp.float32), 0.0)
            return jnp.dot(h, wout_ref[...],
                           preferred_element_type=jnp.float32)

        def x_fwd(src_slot, dst_slot):
            return pltpu.make_async_remote_copy(
                src_ref=xbuf.at[src_slot], dst_ref=xbuf.at[dst_slot],
                send_sem=x_send_sems.at[src_slot],
                recv_sem=x_recv_sems.at[dst_slot],
                device_id=(right,), device_id_type=MESH)

        def x_rcv(slot):
            return pltpu.make_async_remote_copy(
                src_ref=xbuf.at[slot], dst_ref=xbuf.at[slot],
                send_sem=x_send_sems.at[slot],
                recv_sem=x_recv_sems.at[slot],
                device_id=(left,), device_id_type=MESH)

        def acc_snd(slot, nslot):
            return pltpu.make_async_remote_copy(
                src_ref=accbuf.at[slot], dst_ref=accbuf.at[nslot],
                send_sem=acc_send_sems.at[slot],
                recv_sem=acc_recv_sems.at[nslot],
                device_id=(right,), device_id_type=MESH)

        def acc_rcv(slot):
            return pltpu.make_async_remote_copy(
                src_ref=accbuf.at[slot], dst_ref=accbuf.at[slot],
                send_sem=acc_send_sems.at[slot],
                recv_sem=acc_recv_sems.at[slot],
                device_id=(left,), device_id_type=MESH)

        def home_cpy():
            return pltpu.make_async_remote_copy(
                src_ref=accbuf.at[1], dst_ref=homebuf,
                send_sem=acc_send_sems.at[1], recv_sem=home_sem,
                device_id=(right,), device_id_type=MESH)

        def grant(sem):
            pl.semaphore_signal(sem, inc=1, device_id=(left,),
                                device_id_type=MESH)

        barrier = pltpu.get_barrier_semaphore()
        for nbr in (left, right):
            pl.semaphore_signal(barrier, inc=1, device_id=(nbr,),
                                device_id_type=MESH)
        pl.semaphore_wait(barrier, 2)

        xbuf[0] = x_ref[...].astype(jnp.bfloat16)
        f0 = x_fwd(0, 1)
        f0.start()
        cbuf[0] = mlp(0, x_ref[...])
        f0.wait_send()
        grant(x_credit)
        x_rcv(1).wait_recv()
        x_fwd(1, 2).start()

        for l in range(3):

            def step(s, carry, l=l):
                p4 = lax.rem(s, 4)
                n4 = lax.rem(s + 1, 4)
                a4 = lax.rem(s + 2, 4)
                f4 = lax.rem(s + 3, 4)
                p2 = lax.rem(s, 2)
                n2 = lax.rem(s + 1, 2)

                def acc_retire():
                    acc_snd(n2, p2).wait_send()
                    grant(acc_credit)
                if l == 0:
                    pl.when(s >= 1)(acc_retire)
                else:
                    acc_retire()

                @pl.when(s <= N_DEV - 3)
                def _():
                    x_rcv(a4).wait_recv()

                fwd = x_fwd(a4, f4)

                @pl.when(s <= N_DEV - 4)
                def _():
                    if l == 0:
                        @pl.when(s >= 1)
                        def _():
                            pl.semaphore_wait(x_credit, 1)
                    else:
                        pl.semaphore_wait(x_credit, 1)
                    fwd.start()

                def acc_take():
                    pl.semaphore_wait(acc_credit, 1)
                if l == 0:
                    pl.when(s >= 1)(acc_take)
                else:
                    acc_take()

                @pl.when(s >= 1)
                def _():
                    acc_rcv(p2).wait_recv()
                prev = accbuf[p2]
                accbuf[p2] = jnp.where(s == 0, cbuf[p2], prev + cbuf[p2])

                acc_snd(p2, n2).start()

                cbuf[n2] = mlp(l, xbuf[n4])

                @pl.when(s <= N_DEV - 3)
                def _():
                    x_fwd(n4, a4).wait_send()

                def grant_x():
                    grant(x_credit)
                if l < 2:
                    pl.when(s != 27)(grant_x)
                else:
                    pl.when(s <= 26)(grant_x)

                return carry

            lax.fori_loop(0, N_DEV - 1, step, 0)

            acc_snd(0, 1).wait_send()
            acc_rcv(1).wait_recv()
            accbuf[1] = accbuf[1] + cbuf[1]
            home_cpy().start()

            if l < 2:
                hr = home_cpy()
                hr.wait_recv()
                homev = homebuf[...]
                cbuf[0] = mlp(l + 1, homev)
                xbuf[0] = homev.astype(jnp.bfloat16)
                pl.semaphore_wait(x_credit, 1)
                fb = x_fwd(0, 1)
                fb.start()
                fb.wait_send()
                grant(x_credit)
                x_rcv(1).wait_recv()
                pl.semaphore_wait(x_credit, 1)
                x_fwd(1, 2).start()

        home_cpy().wait_recv()
        out_ref[...] = homebuf[...]
        home_cpy().wait_send()

    return pl.pallas_call(
        body,
        out_shape=jax.ShapeDtypeStruct((m, d), jnp.float32),
        in_specs=[pl.BlockSpec(memory_space=pltpu.VMEM)] * 7,
        out_specs=pl.BlockSpec(memory_space=pltpu.VMEM),
        scratch_shapes=[
            pltpu.VMEM((4, m, d), jnp.bfloat16),
            pltpu.VMEM((2, m, d), jnp.float32),
            pltpu.VMEM((2, m, d), jnp.float32),
            pltpu.VMEM((m, d), jnp.float32),
            pltpu.SemaphoreType.DMA((4,)),
            pltpu.SemaphoreType.DMA((4,)),
            pltpu.SemaphoreType.DMA((2,)),
            pltpu.SemaphoreType.DMA((2,)),
            pltpu.SemaphoreType.DMA,
            pltpu.SemaphoreType.REGULAR,
            pltpu.SemaphoreType.REGULAR,
        ],
        compiler_params=pltpu.CompilerParams(
            collective_id=0,
            vmem_limit_bytes=100 * 1024 * 1024,
        ),
    )(x, Win0, Wout0, Win1, Wout1, Win2, Wout2)


# device time: 610579 ns/iter; 1.6445x vs baseline; 1.0956x over previous
import jax
import jax.numpy as jnp
from jax import lax
from jax.experimental import pallas as pl
from jax.experimental.pallas import tpu as pltpu

N_DEV = 32
MESH = pl.DeviceIdType.MESH


def kernel(x, Win0, Wout0, Win1, Wout1, Win2, Wout2):
    m, d = x.shape
    d2 = d // 2

    def body(x_ref, win0_ref, wout0_ref, win1_ref, wout1_ref, win2_ref,
             wout2_ref, out_ref, xbuf, accbuf, cbuf, homebuf, x_send_sems,
             x_recv_sems, acc_send_sems, acc_recv_sems, home_sems, x_credit,
             acc_credit):
        me = lax.axis_index("i")
        left = lax.rem(me - 1 + N_DEV, N_DEV)
        right = lax.rem(me + 1, N_DEV)

        weights = ((win0_ref, wout0_ref), (win1_ref, wout1_ref),
                   (win2_ref, wout2_ref))

        def mlp(l, xval):
            win_ref, wout_ref = weights[l]
            xval = xval.astype(jnp.float32)
            h = jnp.maximum(
                jnp.dot(xval, win_ref[...],
                        preferred_element_type=jnp.float32), 0.0)
            return jnp.dot(h, wout_ref[...],
                           preferred_element_type=jnp.float32)

        def put_c(slot, c):
            cbuf[slot, 0] = c[:, :d2]
            cbuf[slot, 1] = c[:, d2:]

        def x_fwd(src_slot, dst_slot):
            return pltpu.make_async_remote_copy(
                src_ref=xbuf.at[src_slot], dst_ref=xbuf.at[dst_slot],
                send_sem=x_send_sems.at[src_slot],
                recv_sem=x_recv_sems.at[dst_slot],
                device_id=(right,), device_id_type=MESH)

        def x_rcv(slot):
            return pltpu.make_async_remote_copy(
                src_ref=xbuf.at[slot], dst_ref=xbuf.at[slot],
                send_sem=x_send_sems.at[slot],
                recv_sem=x_recv_sems.at[slot],
                device_id=(left,), device_id_type=MESH)

        def acc_snd(slot, h, nslot):
            return pltpu.make_async_remote_copy(
                src_ref=accbuf.at[slot, h], dst_ref=accbuf.at[nslot, h],
                send_sem=acc_send_sems.at[slot, h],
                recv_sem=acc_recv_sems.at[nslot, h],
                device_id=(right,), device_id_type=MESH)

        def acc_rcv(slot, h):
            return pltpu.make_async_remote_copy(
                src_ref=accbuf.at[slot, h], dst_ref=accbuf.at[slot, h],
                send_sem=acc_send_sems.at[slot, h],
                recv_sem=acc_recv_sems.at[slot, h],
                device_id=(left,), device_id_type=MESH)

        def home_cpy(h):
            return pltpu.make_async_remote_copy(
                src_ref=accbuf.at[1, h], dst_ref=homebuf.at[h],
                send_sem=acc_send_sems.at[1, h], recv_sem=home_sems.at[h],
                device_id=(right,), device_id_type=MESH)

        def grant(sem):
            pl.semaphore_signal(sem, inc=1, device_id=(left,),
                                device_id_type=MESH)

        barrier = pltpu.get_barrier_semaphore()
        for nbr in (left, right):
            pl.semaphore_signal(barrier, inc=1, device_id=(nbr,),
                                device_id_type=MESH)
        pl.semaphore_wait(barrier, 2)

        xbuf[0] = x_ref[...].astype(jnp.bfloat16)
        f0 = x_fwd(0, 1)
        f0.start()
        put_c(0, mlp(0, x_ref[...]))
        f0.wait_send()
        grant(x_credit)
        x_rcv(1).wait_recv()
        x_fwd(1, 2).start()

        for l in range(3):

            def step(s, carry, l=l):
                n4 = lax.rem(s + 1, 4)
                a4 = lax.rem(s + 2, 4)
                f4 = lax.rem(s + 3, 4)
                p2 = lax.rem(s, 2)
                n2 = lax.rem(s + 1, 2)

                def acc_retire():
                    acc_snd(n2, 0, p2).wait_send()
                    acc_snd(n2, 1, p2).wait_send()
                    grant(acc_credit)
                if l == 0:
                    pl.when(s >= 1)(acc_retire)
                else:
                    acc_retire()

                @pl.when(s <= N_DEV - 3)
                def _():
                    x_rcv(a4).wait_recv()

                fwd = x_fwd(a4, f4)

                @pl.when(s <= N_DEV - 4)
                def _():
                    if l == 0:
                        @pl.when(s >= 1)
                        def _():
                            pl.semaphore_wait(x_credit, 1)
                    else:
                        pl.semaphore_wait(x_credit, 1)
                    fwd.start()

                def acc_take():
                    pl.semaphore_wait(acc_credit, 1)
                if l == 0:
                    pl.when(s >= 1)(acc_take)
                else:
                    acc_take()

                for h in (0, 1):
                    @pl.when(s >= 1)
                    def _(h=h):
                        acc_rcv(p2, h).wait_recv()
                    prev = accbuf[p2, h]
                    accbuf[p2, h] = jnp.where(s == 0, cbuf[p2, h],
                                              prev + cbuf[p2, h])
                    acc_snd(p2, h, n2).start()

                put_c(n2, mlp(l, xbuf[n4]))

                @pl.when(s <= N_DEV - 3)
                def _():
                    x_fwd(n4, a4).wait_send()

                def grant_x():
                    grant(x_credit)
                if l < 2:
                    pl.when(s != 27)(grant_x)
                else:
                    pl.when(s <= 26)(grant_x)

                return carry

            lax.fori_loop(0, N_DEV - 1, step, 0)

            acc_snd(0, 0, 1).wait_send()
            acc_snd(0, 1, 1).wait_send()
            for h in (0, 1):
                acc_rcv(1, h).wait_recv()
                accbuf[1, h] = accbuf[1, h] + cbuf[1, h]
                home_cpy(h).start()

            if l < 2:
                home_cpy(0).wait_recv()
                home_cpy(1).wait_recv()
                homev = jnp.concatenate([homebuf[0], homebuf[1]], axis=1)
                put_c(0, mlp(l + 1, homev))
                xbuf[0] = homev.astype(jnp.bfloat16)
                pl.semaphore_wait(x_credit, 1)
                fb = x_fwd(0, 1)
                fb.start()
                fb.wait_send()
                grant(x_credit)
                x_rcv(1).wait_recv()
                pl.semaphore_wait(x_credit, 1)
                x_fwd(1, 2).start()

        home_cpy(0).wait_recv()
        home_cpy(1).wait_recv()
        out_ref[:, :d2] = homebuf[0]
        out_ref[:, d2:] = homebuf[1]
        home_cpy(0).wait_send()
        home_cpy(1).wait_send()

    return pl.pallas_call(
        body,
        out_shape=jax.ShapeDtypeStruct((m, d), jnp.float32),
        in_specs=[pl.BlockSpec(memory_space=pltpu.VMEM)] * 7,
        out_specs=pl.BlockSpec(memory_space=pltpu.VMEM),
        scratch_shapes=[
            pltpu.VMEM((4, m, d), jnp.bfloat16),
            pltpu.VMEM((2, 2, m, d2), jnp.float32),
            pltpu.VMEM((2, 2, m, d2), jnp.float32),
            pltpu.VMEM((2, m, d2), jnp.float32),
            pltpu.SemaphoreType.DMA((4,)),
            pltpu.SemaphoreType.DMA((4,)),
            pltpu.SemaphoreType.DMA((2, 2)),
            pltpu.SemaphoreType.DMA((2, 2)),
            pltpu.SemaphoreType.DMA((2,)),
            pltpu.SemaphoreType.REGULAR,
            pltpu.SemaphoreType.REGULAR,
        ],
        compiler_params=pltpu.CompilerParams(
            collective_id=0,
            vmem_limit_bytes=100 * 1024 * 1024,
        ),
    )(x, Win0, Wout0, Win1, Wout1, Win2, Wout2)
